# Initial kernel scaffold; baseline (speedup 1.0000x reference)
import functools

import jax
import jax.numpy as jnp
from jax import lax
from jax.experimental import pallas as pl
from jax.experimental.pallas import tpu as pltpu
from jax.experimental.pallas import tpu_sc as plsc

VOCAB = 100000
MODEL_DIM = 128
BATCH = 1024
SEQ = 200

NC, NS = 2, 16
NW = NC * NS
ROWS = BATCH * SEQ
ROWS_PER_W = ROWS // NW
CHUNK = 100
NCHUNK = ROWS_PER_W // CHUNK


def _pos_table():
    pos = jnp.arange(SEQ, dtype=jnp.float32)[:, None]
    i = jnp.arange(MODEL_DIM)[None, :]
    angle = pos / jnp.power(10000.0, (2 * (i // 2)).astype(jnp.float32) / float(MODEL_DIM))
    sin_v = jnp.sin(angle[:, 0::2])
    cos_v = jnp.cos(angle[:, 1::2])
    pe = jnp.concatenate([sin_v[..., None], cos_v[..., None]], axis=-1)
    return pe.reshape(SEQ, MODEL_DIM)


def _sc_body(idx_hbm, pe_hbm, table_hbm, out_hbm, idx_v, pe_v, rows_v, sem):
    wid = lax.axis_index("s") * NC + lax.axis_index("c")
    pltpu.sync_copy(idx_hbm.at[wid], idx_v)
    pltpu.sync_copy(pe_hbm, pe_v)
    base = wid * ROWS_PER_W

    def chunk_body(c, carry):
        pltpu.async_copy(table_hbm.at[idx_v.at[c]], rows_v, sem).wait()
        pe_off = (c % 2) * CHUNK

        def row_body(r, carry2):
            for k in range(MODEL_DIM // 16):
                sl = pl.ds(k * 16, 16)
                rows_v[r, sl] = rows_v[r, sl] + pe_v[pe_off + r, sl]
            return carry2

        lax.fori_loop(0, CHUNK, row_body, 0)
        pltpu.sync_copy(rows_v, out_hbm.at[pl.ds(base + c * CHUNK, CHUNK)])
        return carry

    lax.fori_loop(0, NCHUNK, chunk_body, 0)


@jax.jit
def kernel(inputs, tok_emb_table):
    idx3 = inputs.reshape(NW, NCHUNK, CHUNK).astype(jnp.int32)
    pe = _pos_table()
    mesh = plsc.VectorSubcoreMesh(core_axis_name="c", subcore_axis_name="s")
    run = functools.partial(
        pl.kernel,
        mesh=mesh,
        out_type=jax.ShapeDtypeStruct((ROWS, MODEL_DIM), jnp.float32),
        scratch_types=[
            pltpu.VMEM((NCHUNK, CHUNK), jnp.int32),
            pltpu.VMEM((SEQ, MODEL_DIM), jnp.float32),
            pltpu.VMEM((CHUNK, MODEL_DIM), jnp.float32),
            pltpu.SemaphoreType.DMA,
        ],
    )(_sc_body)
    out = run(idx3, pe, tok_emb_table)
    return out.reshape(BATCH, SEQ, MODEL_DIM)


# speedup vs baseline: 1.8786x; 1.8786x over previous
import functools

import jax
import jax.numpy as jnp
from jax import lax
from jax.experimental import pallas as pl
from jax.experimental.pallas import tpu as pltpu
from jax.experimental.pallas import tpu_sc as plsc

VOCAB = 100000
MODEL_DIM = 128
BATCH = 1024
SEQ = 200

NC, NS = 2, 16
NW = NC * NS
ROWS = BATCH * SEQ
ROWS_PER_W = ROWS // NW
CHUNK = 128
NCHUNK = ROWS_PER_W // CHUNK


def _pos_table():
    pos = jnp.arange(SEQ, dtype=jnp.float32)[:, None]
    i = jnp.arange(MODEL_DIM)[None, :]
    angle = pos / jnp.power(10000.0, (2 * (i // 2)).astype(jnp.float32) / float(MODEL_DIM))
    sin_v = jnp.sin(angle[:, 0::2])
    cos_v = jnp.cos(angle[:, 1::2])
    pe = jnp.concatenate([sin_v[..., None], cos_v[..., None]], axis=-1)
    return pe.reshape(SEQ, MODEL_DIM)


def _sc_body(idx_hbm, pe_hbm, table_hbm, out_hbm, idx_v, pe_v, rows_v, sem):
    wid = lax.axis_index("s") * NC + lax.axis_index("c")
    pltpu.sync_copy(idx_hbm.at[wid], idx_v)
    pltpu.sync_copy(pe_hbm, pe_v)
    base = wid * ROWS_PER_W

    def chunk_body(c, carry):
        pltpu.async_copy(table_hbm.at[idx_v.at[c]], rows_v, sem).wait()
        pe_off = (c * CHUNK) % SEQ

        def row_body(r, carry2):
            for k in range(MODEL_DIM // 16):
                sl = pl.ds(k * 16, 16)
                rows_v[r, sl] = rows_v[r, sl] + pe_v[pe_off + r, sl]
            return carry2

        lax.fori_loop(0, CHUNK, row_body, 0)
        pltpu.sync_copy(rows_v, out_hbm.at[pl.ds(base + c * CHUNK, CHUNK)])
        return carry

    lax.fori_loop(0, NCHUNK, chunk_body, 0)


@jax.jit
def kernel(inputs, tok_emb_table):
    idx3 = inputs.reshape(NW, NCHUNK, CHUNK).astype(jnp.int32)
    pe1 = _pos_table()
    pe = jnp.concatenate([pe1, pe1], axis=0)
    mesh = plsc.VectorSubcoreMesh(core_axis_name="c", subcore_axis_name="s")
    run = functools.partial(
        pl.kernel,
        mesh=mesh,
        out_type=jax.ShapeDtypeStruct((ROWS, MODEL_DIM), jnp.float32),
        scratch_types=[
            pltpu.VMEM((NCHUNK, CHUNK), jnp.int32),
            pltpu.VMEM((2 * SEQ, MODEL_DIM), jnp.float32),
            pltpu.VMEM((CHUNK, MODEL_DIM), jnp.float32),
            pltpu.SemaphoreType.DMA,
        ],
    )(_sc_body)
    out = run(idx3, pe, tok_emb_table)
    return out.reshape(BATCH, SEQ, MODEL_DIM)


# speedup vs baseline: 2.5809x; 1.3738x over previous
import functools

import jax
import jax.numpy as jnp
from jax import lax
from jax.experimental import pallas as pl
from jax.experimental.pallas import tpu as pltpu
from jax.experimental.pallas import tpu_sc as plsc

VOCAB = 100000
MODEL_DIM = 128
BATCH = 1024
SEQ = 200

NC, NS = 2, 16
NW = NC * NS
ROWS = BATCH * SEQ
ROWS_PER_W = ROWS // NW
CHUNK = 128
NCHUNK = ROWS_PER_W // CHUNK


def _pos_table():
    pos = jnp.arange(SEQ, dtype=jnp.float32)[:, None]
    i = jnp.arange(MODEL_DIM)[None, :]
    angle = pos / jnp.power(10000.0, (2 * (i // 2)).astype(jnp.float32) / float(MODEL_DIM))
    sin_v = jnp.sin(angle[:, 0::2])
    cos_v = jnp.cos(angle[:, 1::2])
    pe = jnp.concatenate([sin_v[..., None], cos_v[..., None]], axis=-1)
    return pe.reshape(SEQ, MODEL_DIM)


def _sc_body(idx_hbm, pe_hbm, table_hbm, out_hbm, idx_v, pe_v,
             rows0, rows1, ob0, ob1, gs0, gs1, ss0, ss1):
    wid = lax.axis_index("s") * NC + lax.axis_index("c")
    pltpu.sync_copy(idx_hbm.at[wid], idx_v)
    pltpu.sync_copy(pe_hbm, pe_v)
    base = wid * ROWS_PER_W
    rows = (rows0, rows1)
    obs = (ob0, ob1)
    gsems = (gs0, gs1)
    ssems = (ss0, ss1)

    pltpu.async_copy(table_hbm.at[idx_v.at[0]], rows0, gs0)
    pltpu.async_copy(table_hbm.at[idx_v.at[1]], rows1, gs1)

    def pair_body(i, carry):
        c = i * 2
        for b in range(2):
            cc = c + b
            r_v, o_v, gs, ss = rows[b], obs[b], gsems[b], ssems[b]
            out_slc = out_hbm.at[pl.ds(base + cc * CHUNK, CHUNK)]
            pltpu.make_async_copy(table_hbm.at[idx_v.at[cc]], r_v, gs).wait()

            @pl.when(cc >= 2)
            def _():
                pltpu.make_async_copy(o_v, out_slc, ss).wait()

            pe_off = (cc * CHUNK) % SEQ

            def row_body(r, carry2):
                for k in range(MODEL_DIM // 16):
                    sl = pl.ds(k * 16, 16)
                    o_v[r, sl] = r_v[r, sl] + pe_v[pe_off + r, sl]
                return carry2

            lax.fori_loop(0, CHUNK, row_body, 0, unroll=2)

            @pl.when(cc + 2 < NCHUNK)
            def _():
                pltpu.async_copy(table_hbm.at[idx_v.at[cc + 2]], r_v, gs)

            pltpu.async_copy(o_v, out_slc, ss)
        return carry

    lax.fori_loop(0, NCHUNK // 2, pair_body, 0)
    pltpu.make_async_copy(ob0, out_hbm.at[pl.ds(base, CHUNK)], ss0).wait()
    pltpu.make_async_copy(ob1, out_hbm.at[pl.ds(base, CHUNK)], ss1).wait()


@jax.jit
def kernel(inputs, tok_emb_table):
    idx3 = inputs.reshape(NW, NCHUNK, CHUNK).astype(jnp.int32)
    pe1 = _pos_table()
    pe = jnp.concatenate([pe1, pe1], axis=0)
    mesh = plsc.VectorSubcoreMesh(core_axis_name="c", subcore_axis_name="s")
    run = functools.partial(
        pl.kernel,
        mesh=mesh,
        out_type=jax.ShapeDtypeStruct((ROWS, MODEL_DIM), jnp.float32),
        scratch_types=[
            pltpu.VMEM((NCHUNK, CHUNK), jnp.int32),
            pltpu.VMEM((2 * SEQ, MODEL_DIM), jnp.float32),
            pltpu.VMEM((CHUNK, MODEL_DIM), jnp.float32),
            pltpu.VMEM((CHUNK, MODEL_DIM), jnp.float32),
            pltpu.VMEM((CHUNK, MODEL_DIM), jnp.float32),
            pltpu.VMEM((CHUNK, MODEL_DIM), jnp.float32),
            pltpu.SemaphoreType.DMA,
            pltpu.SemaphoreType.DMA,
            pltpu.SemaphoreType.DMA,
            pltpu.SemaphoreType.DMA,
        ],
    )(_sc_body)
    out = run(idx3, pe, tok_emb_table)
    return out.reshape(BATCH, SEQ, MODEL_DIM)


# speedup vs baseline: 3.5956x; 1.3932x over previous
import functools

import jax
import jax.numpy as jnp
from jax import lax
from jax.experimental import pallas as pl
from jax.experimental.pallas import tpu as pltpu
from jax.experimental.pallas import tpu_sc as plsc

VOCAB = 100000
MODEL_DIM = 128
BATCH = 1024
SEQ = 200

NC, NS = 2, 16
NW = NC * NS
ROWS = BATCH * SEQ
ROWS_PER_W = ROWS // NW
CHUNK = 64
NCHUNK = ROWS_PER_W // CHUNK
NBUF = 4


def _pos_table():
    pos = jnp.arange(SEQ, dtype=jnp.float32)[:, None]
    i = jnp.arange(MODEL_DIM)[None, :]
    angle = pos / jnp.power(10000.0, (2 * (i // 2)).astype(jnp.float32) / float(MODEL_DIM))
    sin_v = jnp.sin(angle[:, 0::2])
    cos_v = jnp.cos(angle[:, 1::2])
    pe = jnp.concatenate([sin_v[..., None], cos_v[..., None]], axis=-1)
    return pe.reshape(SEQ, MODEL_DIM)


def _sc_body(idx_hbm, pe_hbm, table_hbm, out_hbm, idx_v, pe_v,
             ob0, ob1, ob2, ob3, gs0, gs1, gs2, gs3, ss0, ss1, ss2, ss3):
    wid = lax.axis_index("s") * NC + lax.axis_index("c")
    pltpu.sync_copy(idx_hbm.at[wid], idx_v)
    pltpu.sync_copy(pe_hbm, pe_v)
    base = wid * ROWS_PER_W
    obs = (ob0, ob1, ob2, ob3)
    gsems = (gs0, gs1, gs2, gs3)
    ssems = (ss0, ss1, ss2, ss3)

    def prefill(cc, o_v):
        pe_off = (cc * CHUNK) % SEQ

        def row_body(r, carry2):
            for k in range(MODEL_DIM // 16):
                sl = pl.ds(k * 16, 16)
                o_v[r, sl] = pe_v[pe_off + r, sl]
            return carry2

        lax.fori_loop(0, CHUNK, row_body, 0, unroll=2)

    def gather_add(cc, o_v, gs):
        return pltpu.async_copy(table_hbm.at[idx_v.at[cc]], o_v, gs,
                                add=True)

    def out_slice(cc):
        return out_hbm.at[pl.ds(base + cc * CHUNK, CHUNK)]

    for b in range(2):
        prefill(b, obs[b])
        gather_add(b, obs[b], gsems[b])

    def quad_body(i, carry):
        c = i * NBUF
        for b in range(NBUF):
            cc = c + b
            o_v, gs, ss = obs[b], gsems[b], ssems[b]
            pltpu.make_async_copy(table_hbm.at[idx_v.at[cc]], o_v, gs).wait()
            pltpu.async_copy(o_v, out_slice(cc), ss)

            @pl.when(cc + 2 < NCHUNK)
            def _():
                b2 = (b + 2) % NBUF
                o2 = obs[b2]

                @pl.when(cc >= 2)
                def _():
                    pltpu.make_async_copy(o2, out_slice(cc), ssems[b2]).wait()

                prefill(cc + 2, o2)
                gather_add(cc + 2, o2, gsems[b2])
        return carry

    lax.fori_loop(0, NCHUNK // NBUF, quad_body, 0)
    for b in range(NBUF):
        pltpu.make_async_copy(obs[b], out_hbm.at[pl.ds(base, CHUNK)],
                              ssems[b]).wait()


@jax.jit
def kernel(inputs, tok_emb_table):
    idx3 = inputs.reshape(NW, NCHUNK, CHUNK).astype(jnp.int32)
    pe1 = _pos_table()
    pe = jnp.concatenate([pe1, pe1], axis=0)
    mesh = plsc.VectorSubcoreMesh(core_axis_name="c", subcore_axis_name="s")
    run = functools.partial(
        pl.kernel,
        mesh=mesh,
        out_type=jax.ShapeDtypeStruct((ROWS, MODEL_DIM), jnp.float32),
        scratch_types=[
            pltpu.VMEM((NCHUNK, CHUNK), jnp.int32),
            pltpu.VMEM((2 * SEQ, MODEL_DIM), jnp.float32),
        ] + [pltpu.VMEM((CHUNK, MODEL_DIM), jnp.float32)] * NBUF
          + [pltpu.SemaphoreType.DMA] * (2 * NBUF),
    )(_sc_body)
    out = run(idx3, pe, tok_emb_table)
    return out.reshape(BATCH, SEQ, MODEL_DIM)


# speedup vs baseline: 3.6287x; 1.0092x over previous
import functools

import jax
import jax.numpy as jnp
from jax import lax
from jax.experimental import pallas as pl
from jax.experimental.pallas import tpu as pltpu
from jax.experimental.pallas import tpu_sc as plsc

VOCAB = 100000
MODEL_DIM = 128
BATCH = 1024
SEQ = 200

NC, NS = 2, 16
NW = NC * NS
ROWS = BATCH * SEQ
ROWS_PER_W = ROWS // NW
CHUNK = 64
NCHUNK = ROWS_PER_W // CHUNK
NBUF = 5
SKEW = 3


def _pos_table():
    pos = jnp.arange(SEQ, dtype=jnp.float32)[:, None]
    i = jnp.arange(MODEL_DIM)[None, :]
    angle = pos / jnp.power(10000.0, (2 * (i // 2)).astype(jnp.float32) / float(MODEL_DIM))
    sin_v = jnp.sin(angle[:, 0::2])
    cos_v = jnp.cos(angle[:, 1::2])
    pe = jnp.concatenate([sin_v[..., None], cos_v[..., None]], axis=-1)
    return pe.reshape(SEQ, MODEL_DIM)


def _sc_body(idx_hbm, pe_hbm, table_hbm, out_hbm, idx_v, pe_v,
             ob0, ob1, ob2, ob3, ob4, gs0, gs1, gs2, gs3, gs4,
             ss0, ss1, ss2, ss3, ss4):
    wid = lax.axis_index("s") * NC + lax.axis_index("c")
    pltpu.sync_copy(idx_hbm.at[wid], idx_v)
    pltpu.sync_copy(pe_hbm, pe_v)
    base = wid * ROWS_PER_W
    obs = (ob0, ob1, ob2, ob3, ob4)
    gsems = (gs0, gs1, gs2, gs3, gs4)
    ssems = (ss0, ss1, ss2, ss3, ss4)

    def prefill(cc, o_v):
        pe_off = (cc * CHUNK) % SEQ

        def row_body(r, carry2):
            for k in range(MODEL_DIM // 16):
                sl = pl.ds(k * 16, 16)
                o_v[r, sl] = pe_v[pe_off + r, sl]
            return carry2

        lax.fori_loop(0, CHUNK, row_body, 0, unroll=4)

    def gather_add(cc, o_v, gs):
        return pltpu.async_copy(table_hbm.at[idx_v.at[cc]], o_v, gs,
                                add=True)

    def out_slice(cc):
        return out_hbm.at[pl.ds(base + cc * CHUNK, CHUNK)]

    for b in range(SKEW):
        prefill(b, obs[b])
        gather_add(b, obs[b], gsems[b])

    def ring_body(i, carry):
        c = i * NBUF
        for b in range(NBUF):
            cc = c + b
            o_v, gs, ss = obs[b], gsems[b], ssems[b]
            pltpu.make_async_copy(table_hbm.at[idx_v.at[cc]], o_v, gs).wait()
            pltpu.async_copy(o_v, out_slice(cc), ss)

            @pl.when(cc + SKEW < NCHUNK)
            def _():
                b2 = (b + SKEW) % NBUF
                o2 = obs[b2]

                @pl.when(cc + SKEW >= NBUF)
                def _():
                    pltpu.make_async_copy(o2, out_slice(cc), ssems[b2]).wait()

                prefill(cc + SKEW, o2)
                gather_add(cc + SKEW, o2, gsems[b2])
        return carry

    lax.fori_loop(0, NCHUNK // NBUF, ring_body, 0)
    for b in range(NBUF):
        pltpu.make_async_copy(obs[b], out_hbm.at[pl.ds(base, CHUNK)],
                              ssems[b]).wait()


@jax.jit
def kernel(inputs, tok_emb_table):
    idx3 = inputs.reshape(NW, NCHUNK, CHUNK).astype(jnp.int32)
    pe1 = _pos_table()
    pe = jnp.concatenate([pe1, pe1], axis=0)
    mesh = plsc.VectorSubcoreMesh(core_axis_name="c", subcore_axis_name="s")
    run = functools.partial(
        pl.kernel,
        mesh=mesh,
        out_type=jax.ShapeDtypeStruct((ROWS, MODEL_DIM), jnp.float32),
        scratch_types=[
            pltpu.VMEM((NCHUNK, CHUNK), jnp.int32),
            pltpu.VMEM((2 * SEQ, MODEL_DIM), jnp.float32),
        ] + [pltpu.VMEM((CHUNK, MODEL_DIM), jnp.float32)] * NBUF
          + [pltpu.SemaphoreType.DMA] * (2 * NBUF),
    )(_sc_body)
    out = run(idx3, pe, tok_emb_table)
    return out.reshape(BATCH, SEQ, MODEL_DIM)


# speedup vs baseline: 7.1751x; 1.9773x over previous
import functools

import jax
import jax.numpy as jnp
from jax import lax
from jax.experimental import pallas as pl
from jax.experimental.pallas import tpu as pltpu
from jax.experimental.pallas import tpu_sc as plsc

VOCAB = 100000
MODEL_DIM = 128
BATCH = 1024
SEQ = 200

NC, NS = 2, 16
NW = NC * NS
ROWS = BATCH * SEQ
ROWS_PER_W = ROWS // NW
CHUNK = 64
NCHUNK = ROWS_PER_W // CHUNK
NBUF = 5
SKEW = 3


def _pos_table():
    pos = jnp.arange(SEQ, dtype=jnp.float32)[:, None]
    i = jnp.arange(MODEL_DIM)[None, :]
    angle = pos / jnp.power(10000.0, (2 * (i // 2)).astype(jnp.float32) / float(MODEL_DIM))
    sin_v = jnp.sin(angle[:, 0::2])
    cos_v = jnp.cos(angle[:, 1::2])
    pe = jnp.concatenate([sin_v[..., None], cos_v[..., None]], axis=-1)
    return pe.reshape(SEQ, MODEL_DIM)


def _sc_body(idx_hbm, pe_hbm, table_hbm, out_hbm, idx_v, pe_sh,
             ob0, ob1, ob2, ob3, ob4, gs0, gs1, gs2, gs3, gs4,
             ss0, ss1, ss2, ss3, ss4, ps0, ps1, ps2, ps3, ps4):
    sid = lax.axis_index("s")
    wid = sid * NC + lax.axis_index("c")
    pltpu.sync_copy(idx_hbm.at[wid], idx_v)

    @pl.when(sid == 0)
    def _():
        pltpu.sync_copy(pe_hbm, pe_sh)

    plsc.subcore_barrier()

    base = wid * ROWS_PER_W
    obs = (ob0, ob1, ob2, ob3, ob4)
    gsems = (gs0, gs1, gs2, gs3, gs4)
    ssems = (ss0, ss1, ss2, ss3, ss4)
    psems = (ps0, ps1, ps2, ps3, ps4)

    def pe_slice(cc):
        return pe_sh.at[pl.ds((cc * CHUNK) % SEQ, CHUNK)]

    def pe_fill(cc, bb):
        pltpu.async_copy(pe_slice(cc), obs[bb], psems[bb])

    def gather_add(cc, bb):
        pltpu.async_copy(table_hbm.at[idx_v.at[cc]], obs[bb], gsems[bb],
                         add=True)

    def out_slice(cc):
        return out_hbm.at[pl.ds(base + cc * CHUNK, CHUNK)]

    for b in range(SKEW):
        pe_fill(b, b)
    for b in range(SKEW - 1):
        pltpu.make_async_copy(pe_slice(b), obs[b], psems[b]).wait()
        gather_add(b, b)

    def ring_body(i, carry):
        c = i * NBUF
        for b in range(NBUF):
            cc = c + b
            o_v, gs, ss = obs[b], gsems[b], ssems[b]
            pltpu.make_async_copy(table_hbm.at[idx_v.at[cc]], o_v, gs).wait()
            pltpu.async_copy(o_v, out_slice(cc), ss)

            @pl.when(cc + SKEW < NCHUNK)
            def _():
                b3 = (b + SKEW) % NBUF

                @pl.when(cc + SKEW >= NBUF)
                def _():
                    pltpu.make_async_copy(obs[b3], out_slice(cc),
                                          ssems[b3]).wait()

                pe_fill(cc + SKEW, b3)

            @pl.when(cc + SKEW - 1 < NCHUNK)
            def _():
                b2 = (b + SKEW - 1) % NBUF
                pltpu.make_async_copy(pe_slice(cc), obs[b2],
                                      psems[b2]).wait()
                gather_add(cc + SKEW - 1, b2)
        return carry

    lax.fori_loop(0, NCHUNK // NBUF, ring_body, 0)
    for b in range(NBUF):
        pltpu.make_async_copy(obs[b], out_hbm.at[pl.ds(base, CHUNK)],
                              ssems[b]).wait()


@jax.jit
def kernel(inputs, tok_emb_table):
    idx3 = inputs.reshape(NW, NCHUNK, CHUNK).astype(jnp.int32)
    pe1 = _pos_table()
    pe = jnp.concatenate([pe1, pe1], axis=0)
    mesh = plsc.VectorSubcoreMesh(core_axis_name="c", subcore_axis_name="s")
    run = functools.partial(
        pl.kernel,
        mesh=mesh,
        out_type=jax.ShapeDtypeStruct((ROWS, MODEL_DIM), jnp.float32),
        scratch_types=[
            pltpu.VMEM((NCHUNK, CHUNK), jnp.int32),
            pltpu.VMEM_SHARED((2 * SEQ, MODEL_DIM), jnp.float32),
        ] + [pltpu.VMEM((CHUNK, MODEL_DIM), jnp.float32)] * NBUF
          + [pltpu.SemaphoreType.DMA] * (3 * NBUF),
    )(_sc_body)
    out = run(idx3, pe, tok_emb_table)
    return out.reshape(BATCH, SEQ, MODEL_DIM)


# speedup vs baseline: 7.7195x; 1.0759x over previous
import functools

import jax
import jax.numpy as jnp
from jax import lax
from jax.experimental import pallas as pl
from jax.experimental.pallas import tpu as pltpu
from jax.experimental.pallas import tpu_sc as plsc

VOCAB = 100000
MODEL_DIM = 128
BATCH = 1024
SEQ = 200

NC, NS = 2, 16
NW = NC * NS
ROWS = BATCH * SEQ
ROWS_PER_W = ROWS // NW
CHUNK = 128
NCHUNK = ROWS_PER_W // CHUNK
NBUF = 5
SKEW = 3


def _pos_table():
    pos = jnp.arange(SEQ, dtype=jnp.float32)[:, None]
    i = jnp.arange(MODEL_DIM)[None, :]
    angle = pos / jnp.power(10000.0, (2 * (i // 2)).astype(jnp.float32) / float(MODEL_DIM))
    sin_v = jnp.sin(angle[:, 0::2])
    cos_v = jnp.cos(angle[:, 1::2])
    pe = jnp.concatenate([sin_v[..., None], cos_v[..., None]], axis=-1)
    return pe.reshape(SEQ, MODEL_DIM)


def _sc_body(idx_hbm, pe_hbm, table_hbm, out_hbm, idx_v, pe_sh,
             ob0, ob1, ob2, ob3, ob4, gs0, gs1, gs2, gs3, gs4,
             ss0, ss1, ss2, ss3, ss4, ps0, ps1, ps2, ps3, ps4):
    sid = lax.axis_index("s")
    wid = sid * NC + lax.axis_index("c")
    pltpu.sync_copy(idx_hbm.at[wid], idx_v)

    @pl.when(sid == 0)
    def _():
        pltpu.sync_copy(pe_hbm, pe_sh)

    plsc.subcore_barrier()

    base = wid * ROWS_PER_W
    obs = (ob0, ob1, ob2, ob3, ob4)
    gsems = (gs0, gs1, gs2, gs3, gs4)
    ssems = (ss0, ss1, ss2, ss3, ss4)
    psems = (ps0, ps1, ps2, ps3, ps4)

    def pe_slice(cc):
        return pe_sh.at[pl.ds((cc * CHUNK) % SEQ, CHUNK)]

    def pe_fill(cc, bb):
        pltpu.async_copy(pe_slice(cc), obs[bb], psems[bb])

    def gather_add(cc, bb):
        pltpu.async_copy(table_hbm.at[idx_v.at[cc]], obs[bb], gsems[bb],
                         add=True)

    def out_slice(cc):
        return out_hbm.at[pl.ds(base + cc * CHUNK, CHUNK)]

    for b in range(SKEW):
        pe_fill(b, b)
    for b in range(SKEW - 1):
        pltpu.make_async_copy(pe_slice(b), obs[b], psems[b]).wait()
        gather_add(b, b)

    def ring_body(i, carry):
        c = i * NBUF
        for b in range(NBUF):
            cc = c + b
            o_v, gs, ss = obs[b], gsems[b], ssems[b]
            pltpu.make_async_copy(table_hbm.at[idx_v.at[cc]], o_v, gs).wait()
            pltpu.async_copy(o_v, out_slice(cc), ss)

            @pl.when(cc + SKEW < NCHUNK)
            def _():
                b3 = (b + SKEW) % NBUF

                @pl.when(cc + SKEW >= NBUF)
                def _():
                    pltpu.make_async_copy(obs[b3], out_slice(cc),
                                          ssems[b3]).wait()

                pe_fill(cc + SKEW, b3)

            @pl.when(cc + SKEW - 1 < NCHUNK)
            def _():
                b2 = (b + SKEW - 1) % NBUF
                pltpu.make_async_copy(pe_slice(cc), obs[b2],
                                      psems[b2]).wait()
                gather_add(cc + SKEW - 1, b2)
        return carry

    lax.fori_loop(0, NCHUNK // NBUF, ring_body, 0)
    for b in range(NBUF):
        pltpu.make_async_copy(obs[b], out_hbm.at[pl.ds(base, CHUNK)],
                              ssems[b]).wait()


@jax.jit
def kernel(inputs, tok_emb_table):
    idx3 = inputs.reshape(NW, NCHUNK, CHUNK).astype(jnp.int32)
    pe1 = _pos_table()
    pe = jnp.concatenate([pe1, pe1], axis=0)
    mesh = plsc.VectorSubcoreMesh(core_axis_name="c", subcore_axis_name="s")
    run = functools.partial(
        pl.kernel,
        mesh=mesh,
        out_type=jax.ShapeDtypeStruct((ROWS, MODEL_DIM), jnp.float32),
        scratch_types=[
            pltpu.VMEM((NCHUNK, CHUNK), jnp.int32),
            pltpu.VMEM_SHARED((2 * SEQ, MODEL_DIM), jnp.float32),
        ] + [pltpu.VMEM((CHUNK, MODEL_DIM), jnp.float32)] * NBUF
          + [pltpu.SemaphoreType.DMA] * (3 * NBUF),
    )(_sc_body)
    out = run(idx3, pe, tok_emb_table)
    return out.reshape(BATCH, SEQ, MODEL_DIM)


# speedup vs baseline: 7.7285x; 1.0012x over previous
import functools

import jax
import jax.numpy as jnp
from jax import lax
from jax.experimental import pallas as pl
from jax.experimental.pallas import tpu as pltpu
from jax.experimental.pallas import tpu_sc as plsc

VOCAB = 100000
MODEL_DIM = 128
BATCH = 1024
SEQ = 200

NC, NS = 2, 16
NW = NC * NS
ROWS = BATCH * SEQ
ROWS_PER_W = ROWS // NW
CHUNK = 128
NCHUNK = ROWS_PER_W // CHUNK
NBUF = 5
SKEW = 4


def _pos_table():
    pos = jnp.arange(SEQ, dtype=jnp.float32)[:, None]
    i = jnp.arange(MODEL_DIM)[None, :]
    angle = pos / jnp.power(10000.0, (2 * (i // 2)).astype(jnp.float32) / float(MODEL_DIM))
    sin_v = jnp.sin(angle[:, 0::2])
    cos_v = jnp.cos(angle[:, 1::2])
    pe = jnp.concatenate([sin_v[..., None], cos_v[..., None]], axis=-1)
    return pe.reshape(SEQ, MODEL_DIM)


def _sc_body(idx_hbm, pe_hbm, table_hbm, out_hbm, idx_v, pe_sh,
             ob0, ob1, ob2, ob3, ob4, gs0, gs1, gs2, gs3, gs4,
             ss0, ss1, ss2, ss3, ss4, ps0, ps1, ps2, ps3, ps4):
    sid = lax.axis_index("s")
    wid = sid * NC + lax.axis_index("c")
    pltpu.sync_copy(idx_hbm.at[wid], idx_v)

    @pl.when(sid == 0)
    def _():
        pltpu.sync_copy(pe_hbm, pe_sh)

    plsc.subcore_barrier()

    base = wid * ROWS_PER_W
    obs = (ob0, ob1, ob2, ob3, ob4)
    gsems = (gs0, gs1, gs2, gs3, gs4)
    ssems = (ss0, ss1, ss2, ss3, ss4)
    psems = (ps0, ps1, ps2, ps3, ps4)

    def pe_slice(cc):
        return pe_sh.at[pl.ds((cc * CHUNK) % SEQ, CHUNK)]

    def pe_fill(cc, bb):
        pltpu.async_copy(pe_slice(cc), obs[bb], psems[bb])

    def gather_add(cc, bb):
        pltpu.async_copy(table_hbm.at[idx_v.at[cc]], obs[bb], gsems[bb],
                         add=True)

    def out_slice(cc):
        return out_hbm.at[pl.ds(base + cc * CHUNK, CHUNK)]

    for b in range(SKEW):
        pe_fill(b, b)
    for b in range(SKEW - 1):
        pltpu.make_async_copy(pe_slice(b), obs[b], psems[b]).wait()
        gather_add(b, b)

    def ring_body(i, carry):
        c = i * NBUF
        for b in range(NBUF):
            cc = c + b
            o_v, gs, ss = obs[b], gsems[b], ssems[b]
            pltpu.make_async_copy(table_hbm.at[idx_v.at[cc]], o_v, gs).wait()
            pltpu.async_copy(o_v, out_slice(cc), ss)

            @pl.when(cc + SKEW < NCHUNK)
            def _():
                b3 = (b + SKEW) % NBUF

                @pl.when(cc + SKEW >= NBUF)
                def _():
                    pltpu.make_async_copy(obs[b3], out_slice(cc),
                                          ssems[b3]).wait()

                pe_fill(cc + SKEW, b3)

            @pl.when(cc + SKEW - 1 < NCHUNK)
            def _():
                b2 = (b + SKEW - 1) % NBUF
                pltpu.make_async_copy(pe_slice(cc), obs[b2],
                                      psems[b2]).wait()
                gather_add(cc + SKEW - 1, b2)
        return carry

    lax.fori_loop(0, NCHUNK // NBUF, ring_body, 0)
    for b in range(NBUF):
        pltpu.make_async_copy(obs[b], out_hbm.at[pl.ds(base, CHUNK)],
                              ssems[b]).wait()


@jax.jit
def kernel(inputs, tok_emb_table):
    idx3 = inputs.reshape(NW, NCHUNK, CHUNK).astype(jnp.int32)
    pe1 = _pos_table()
    pe = jnp.concatenate([pe1, pe1], axis=0)
    mesh = plsc.VectorSubcoreMesh(core_axis_name="c", subcore_axis_name="s")
    run = functools.partial(
        pl.kernel,
        mesh=mesh,
        out_type=jax.ShapeDtypeStruct((ROWS, MODEL_DIM), jnp.float32),
        scratch_types=[
            pltpu.VMEM((NCHUNK, CHUNK), jnp.int32),
            pltpu.VMEM_SHARED((2 * SEQ, MODEL_DIM), jnp.float32),
        ] + [pltpu.VMEM((CHUNK, MODEL_DIM), jnp.float32)] * NBUF
          + [pltpu.SemaphoreType.DMA] * (3 * NBUF),
    )(_sc_body)
    out = run(idx3, pe, tok_emb_table)
    return out.reshape(BATCH, SEQ, MODEL_DIM)
